# Initial kernel scaffold; baseline (speedup 1.0000x reference)
import jax
import jax.numpy as jnp
from jax.experimental import pallas as pl


def kernel(x, Wr, W1, b1, W2, b2):
    raise NotImplementedError("write your pallas kernel here")


# speedup vs baseline: 2.6690x; 2.6690x over previous
import functools

import jax
import jax.numpy as jnp
from jax import lax
from jax.experimental import pallas as pl
from jax.experimental.pallas import tpu as pltpu

S = 2048
H = 1024
DFF = 2048
E = 8
EPAD = 128
TOPK = 2
AUX_COEF = 0.01
Z_COEF = 0.001

SBLK = 512
NSB = S // SBLK


def _routing_body(x_ref, wr_ref, gates_ref, aux_ref):
    x = x_ref[...]
    wr = wr_ref[...]
    logits = lax.dot_general(x, wr, (((1,), (1,)), ((), ())),
                             preferred_element_type=jnp.float32)
    lane = lax.broadcasted_iota(jnp.int32, (S, EPAD), 1)
    valid = lane < E
    neg = jnp.float32(-1e30)
    logits = jnp.where(valid, logits, neg)

    lmax = jnp.max(logits, axis=1, keepdims=True)
    ex = jnp.exp(logits - lmax)
    ssum = jnp.sum(ex, axis=1, keepdims=True)
    probs = ex / ssum

    m1 = jnp.max(probs, axis=1, keepdims=True)
    a1 = jnp.min(jnp.where(probs == m1, lane, EPAD), axis=1, keepdims=True)
    probs2 = jnp.where(lane == a1, neg, probs)
    m2 = jnp.max(probs2, axis=1, keepdims=True)
    a2 = jnp.min(jnp.where(probs2 == m2, lane, EPAD), axis=1, keepdims=True)

    denom = m1 + m2
    g1 = m1 / denom
    g2 = m2 / denom
    oh1 = (lane == a1).astype(jnp.float32)
    oh2 = (lane == a2).astype(jnp.float32)
    gates_ref[...] = oh1 * g1 + oh2 * g2

    tpe = jnp.sum(oh1 + oh2, axis=0)
    fraction = tpe / jnp.float32(S * TOPK)
    mean_prob = jnp.sum(probs, axis=0) / jnp.float32(S)
    lb = jnp.float32(E) * jnp.sum(fraction * mean_prob)
    lse = jnp.log(ssum) + lmax
    z = jnp.sum(lse * lse) / jnp.float32(S)
    aux_ref[...] = jnp.reshape(AUX_COEF * lb + Z_COEF * z, (1, 1))


def _ffn_body(x_ref, gates_ref, w1_ref, b1_ref, w2_ref, b2_ref, out_ref):
    e = pl.program_id(0)
    s = pl.program_id(1)

    @pl.when(jnp.logical_and(e == 0, s == 0))
    def _init():
        out_ref[...] = jnp.zeros_like(out_ref)

    xb = x_ref[pl.ds(s * SBLK, SBLK), :]
    w1 = w1_ref[0]
    w2 = w2_ref[0]
    h = lax.dot_general(xb, w1, (((1,), (1,)), ((), ())),
                        preferred_element_type=jnp.float32)
    h = h + b1_ref[0]
    h = 0.5 * h * (1.0 + lax.erf(h * jnp.float32(0.7071067811865476)))
    y = lax.dot_general(h, w2, (((1,), (1,)), ((), ())),
                        preferred_element_type=jnp.float32)
    y = y + b2_ref[0]
    gb = gates_ref[pl.ds(s * SBLK, SBLK), :]
    lane = lax.broadcasted_iota(jnp.int32, (SBLK, EPAD), 1)
    g = jnp.sum(jnp.where(lane == e, gb, 0.0), axis=1, keepdims=True)
    out_ref[pl.ds(s * SBLK, SBLK), :] += y * g


@jax.jit
def _moe(x, Wr, W1, b1, W2, b2):
    x2d = x.reshape(S, H)
    wr_pad = jnp.zeros((EPAD, H), jnp.float32).at[:E].set(Wr)

    gates, aux = pl.pallas_call(
        _routing_body,
        out_shape=(
            jax.ShapeDtypeStruct((S, EPAD), jnp.float32),
            jax.ShapeDtypeStruct((1, 1), jnp.float32),
        ),
        in_specs=[
            pl.BlockSpec((S, H), lambda: (0, 0)),
            pl.BlockSpec((EPAD, H), lambda: (0, 0)),
        ],
        out_specs=(
            pl.BlockSpec((S, EPAD), lambda: (0, 0)),
            pl.BlockSpec((1, 1), lambda: (0, 0)),
        ),
    )(x2d, wr_pad)

    out = pl.pallas_call(
        _ffn_body,
        grid=(E, NSB),
        out_shape=jax.ShapeDtypeStruct((S, H), jnp.float32),
        in_specs=[
            pl.BlockSpec((S, H), lambda e, s: (0, 0)),
            pl.BlockSpec((S, EPAD), lambda e, s: (0, 0)),
            pl.BlockSpec((1, DFF, H), lambda e, s: (e, 0, 0)),
            pl.BlockSpec((1, 1, DFF), lambda e, s: (e, 0, 0)),
            pl.BlockSpec((1, H, DFF), lambda e, s: (e, 0, 0)),
            pl.BlockSpec((1, 1, H), lambda e, s: (e, 0, 0)),
        ],
        out_specs=pl.BlockSpec((S, H), lambda e, s: (0, 0)),
    )(x2d, gates, W1, b1.reshape(E, 1, DFF), W2, b2.reshape(E, 1, H))

    return out.reshape(1, S, H), aux[0, 0]


def kernel(x, Wr, W1, b1, W2, b2):
    return _moe(x, Wr, W1, b1, W2, b2)
